# baseline (speedup 1.0000x reference)
import jax
import jax.numpy as jnp
from jax import lax
from jax.experimental import pallas as pl
from jax.experimental.pallas import tpu as pltpu

_N_MAX = 4096.0
_THRESH = 0.5
_H_BLK = 16


def _reduce_kernel(x_ref, out_ref, mask_ref, carry_ref):
    j = pl.program_id(1)

    @pl.when(j == 0)
    def _():
        carry_ref[0] = 0.0

    c0 = carry_ref[0]
    x = x_ref[0]
    hb, w, _ = x.shape
    flags = (x[:, :, 0] > _THRESH).astype(jnp.float32)

    ik = lax.broadcasted_iota(jnp.int32, (w, w), 0)
    jk = lax.broadcasted_iota(jnp.int32, (w, w), 1)
    upper = (ik <= jk).astype(jnp.float32)
    row_cs = jnp.dot(flags, upper, preferred_element_type=jnp.float32)

    row_tot = row_cs[:, w - 1:w]
    ir = lax.broadcasted_iota(jnp.int32, (hb, hb), 0)
    jr = lax.broadcasted_iota(jnp.int32, (hb, hb), 1)
    lower = (jr < ir).astype(jnp.float32)
    row_off = jnp.dot(lower, row_tot, preferred_element_type=jnp.float32)

    count = row_cs + row_off + c0
    m = flags * (count <= _N_MAX).astype(jnp.float32)

    mask_ref[0] = m
    out_ref[0] = x * m[:, :, None]
    carry_ref[0] = c0 + jnp.sum(flags)


def kernel(inputs):
    b, h, w, c = inputs.shape
    grid = (b, h // _H_BLK)
    out, mask = pl.pallas_call(
        _reduce_kernel,
        grid=grid,
        in_specs=[
            pl.BlockSpec((1, _H_BLK, w, c), lambda bi, ji: (bi, ji, 0, 0)),
        ],
        out_specs=[
            pl.BlockSpec((1, _H_BLK, w, c), lambda bi, ji: (bi, ji, 0, 0)),
            pl.BlockSpec((1, _H_BLK, w), lambda bi, ji: (bi, ji, 0)),
        ],
        out_shape=[
            jax.ShapeDtypeStruct((b, h, w, c), inputs.dtype),
            jax.ShapeDtypeStruct((b, h, w), inputs.dtype),
        ],
        scratch_shapes=[pltpu.SMEM((1,), jnp.float32)],
        compiler_params=pltpu.CompilerParams(
            dimension_semantics=("parallel", "arbitrary"),
        ),
    )(inputs)
    return (out, mask.reshape(b, h, w, 1))
